# baseline (speedup 1.0000x reference)
import jax
import jax.numpy as jnp
from jax import lax
from jax.experimental import pallas as pl
from jax.experimental.pallas import tpu as pltpu

_FB = 512


def _vq_body(x_ref, cb_ref, cbhi_ref, cblo_ref, q_ref, codes_ref, loss_ref):
    xb = x_ref[0]
    cb = cb_ref[...]

    x_sq = jnp.sum(xb * xb, axis=0)
    c_sq = jnp.sum(cb * cb, axis=1)

    mm = lax.dot_general(cb, xb, (((1,), (0,)), ((), ())),
                         preferred_element_type=jnp.float32)
    d = (x_sq[None, :] - 2.0 * mm) + c_sq[:, None]

    mins = jnp.min(d, axis=0, keepdims=True)
    iota_c = lax.broadcasted_iota(jnp.int32, d.shape, 0)
    cand = jnp.where(d == mins, iota_c, jnp.int32(2 ** 30))
    codes = jnp.min(cand, axis=0)

    onehot = (cand == codes[None, :]).astype(jnp.bfloat16)
    dn = (((0,), (0,)), ((), ()))
    q_hi = lax.dot_general(cbhi_ref[...], onehot, dn,
                           preferred_element_type=jnp.float32)
    q_lo = lax.dot_general(cblo_ref[...], onehot, dn,
                           preferred_element_type=jnp.float32)
    q = q_hi + q_lo * (1.0 / 512.0)

    q_ref[0] = q
    codes_ref[...] = codes.reshape(1, 1, codes.shape[0])
    loss_ref[...] = jnp.broadcast_to(jnp.sum(mins), (1, 1, 8, 128))


def kernel(x, codebook):
    batch, dim, frames = x.shape
    ncodes = codebook.shape[0]
    nf = frames // _FB

    hi32 = lax.bitcast_convert_type(
        lax.bitcast_convert_type(codebook, jnp.uint32) & jnp.uint32(0xFFFF0000),
        jnp.float32)
    cb_hi = hi32.astype(jnp.bfloat16)
    cb_lo = ((codebook - hi32) * 512.0).astype(jnp.bfloat16)

    q, codes3, lossp = pl.pallas_call(
        _vq_body,
        grid=(batch, nf),
        in_specs=[
            pl.BlockSpec((1, dim, _FB), lambda b, f: (b, 0, f)),
            pl.BlockSpec((ncodes, dim), lambda b, f: (0, 0)),
            pl.BlockSpec((ncodes, dim), lambda b, f: (0, 0)),
            pl.BlockSpec((ncodes, dim), lambda b, f: (0, 0)),
        ],
        out_specs=[
            pl.BlockSpec((1, dim, _FB), lambda b, f: (b, 0, f)),
            pl.BlockSpec((1, 1, _FB), lambda b, f: (b, 0, f)),
            pl.BlockSpec((1, 1, 8, 128), lambda b, f: (b, f, 0, 0)),
        ],
        out_shape=[
            jax.ShapeDtypeStruct((batch, dim, frames), jnp.float32),
            jax.ShapeDtypeStruct((batch, 1, frames), jnp.int32),
            jax.ShapeDtypeStruct((batch, nf, 8, 128), jnp.float32),
        ],
        compiler_params=pltpu.CompilerParams(
            dimension_semantics=("parallel", "parallel"),
        ),
    )(x, codebook, cb_hi, cb_lo)

    codes = codes3.reshape(batch, frames)
    vq_loss = 1.25 * jnp.sum(lossp[:, :, 0, 0]) / (batch * dim * frames)
    return (q, codes, vq_loss)


# speedup vs baseline: 1.1314x; 1.1314x over previous
import jax
import jax.numpy as jnp
from jax import lax
from jax.experimental import pallas as pl
from jax.experimental.pallas import tpu as pltpu

_FB = 1024


def _vq_body(x_ref, cb_ref, cbhi_ref, cblo_ref, q_ref, codes_ref, loss_ref):
    xb = x_ref[0]
    cb = cb_ref[...]

    x_sq = jnp.sum(xb * xb, axis=0)
    c_sq = jnp.sum(cb * cb, axis=1)

    mm = lax.dot_general(cb, xb, (((1,), (0,)), ((), ())),
                         preferred_element_type=jnp.float32)
    d = (x_sq[None, :] - 2.0 * mm) + c_sq[:, None]

    mins = jnp.min(d, axis=0, keepdims=True)
    iota_c = lax.broadcasted_iota(jnp.int32, d.shape, 0)
    cand = jnp.where(d == mins, iota_c, jnp.int32(2 ** 30))
    codes = jnp.min(cand, axis=0)

    onehot = (cand == codes[None, :]).astype(jnp.bfloat16)
    dn = (((0,), (0,)), ((), ()))
    q_hi = lax.dot_general(cbhi_ref[...], onehot, dn,
                           preferred_element_type=jnp.float32)
    q_lo = lax.dot_general(cblo_ref[...], onehot, dn,
                           preferred_element_type=jnp.float32)
    q = q_hi + q_lo * (1.0 / 512.0)

    q_ref[0] = q
    codes_ref[...] = codes.reshape(1, 1, codes.shape[0])
    loss_ref[...] = jnp.broadcast_to(jnp.sum(mins), (1, 1, 8, 128))


def kernel(x, codebook):
    batch, dim, frames = x.shape
    ncodes = codebook.shape[0]
    nf = frames // _FB

    hi32 = lax.bitcast_convert_type(
        lax.bitcast_convert_type(codebook, jnp.uint32) & jnp.uint32(0xFFFF0000),
        jnp.float32)
    cb_hi = hi32.astype(jnp.bfloat16)
    cb_lo = ((codebook - hi32) * 512.0).astype(jnp.bfloat16)

    q, codes3, lossp = pl.pallas_call(
        _vq_body,
        grid=(batch, nf),
        in_specs=[
            pl.BlockSpec((1, dim, _FB), lambda b, f: (b, 0, f)),
            pl.BlockSpec((ncodes, dim), lambda b, f: (0, 0)),
            pl.BlockSpec((ncodes, dim), lambda b, f: (0, 0)),
            pl.BlockSpec((ncodes, dim), lambda b, f: (0, 0)),
        ],
        out_specs=[
            pl.BlockSpec((1, dim, _FB), lambda b, f: (b, 0, f)),
            pl.BlockSpec((1, 1, _FB), lambda b, f: (b, 0, f)),
            pl.BlockSpec((1, 1, 8, 128), lambda b, f: (b, f, 0, 0)),
        ],
        out_shape=[
            jax.ShapeDtypeStruct((batch, dim, frames), jnp.float32),
            jax.ShapeDtypeStruct((batch, 1, frames), jnp.int32),
            jax.ShapeDtypeStruct((batch, nf, 8, 128), jnp.float32),
        ],
        compiler_params=pltpu.CompilerParams(
            dimension_semantics=("parallel", "parallel"),
        ),
    )(x, codebook, cb_hi, cb_lo)

    codes = codes3.reshape(batch, frames)
    vq_loss = 1.25 * jnp.sum(lossp[:, :, 0, 0]) / (batch * dim * frames)
    return (q, codes, vq_loss)


# speedup vs baseline: 1.1753x; 1.0388x over previous
import jax
import jax.numpy as jnp
from jax import lax
from jax.experimental import pallas as pl
from jax.experimental.pallas import tpu as pltpu

_FB = 1024


def _vq_body(x_ref, cb_ref, cbcat_ref, q_ref, codes_ref, loss_ref):
    xb = x_ref[0]
    cb = cb_ref[...]

    x_sq = jnp.sum(xb * xb, axis=0)
    c_sq = jnp.sum(cb * cb, axis=1)

    mm = lax.dot_general(cb, xb, (((1,), (0,)), ((), ())),
                         preferred_element_type=jnp.float32)
    d = (x_sq[None, :] - 2.0 * mm) + c_sq[:, None]

    mins = jnp.min(d, axis=0, keepdims=True)
    iota_c = lax.broadcasted_iota(jnp.int32, d.shape, 0)
    cand = jnp.where(d == mins, iota_c, jnp.int32(2 ** 30))
    codes = jnp.min(cand, axis=0)

    onehot = (cand == codes[None, :]).astype(jnp.bfloat16)
    dn = (((0,), (0,)), ((), ()))
    qq = lax.dot_general(cbcat_ref[...], onehot, dn,
                         preferred_element_type=jnp.float32)
    dim = qq.shape[0] // 2
    q = qq[:dim] + qq[dim:] * (1.0 / 512.0)

    q_ref[0] = q
    codes_ref[...] = codes.reshape(1, 1, codes.shape[0])
    loss_ref[...] = jnp.broadcast_to(jnp.sum(mins), (1, 1, 8, 128))


def kernel(x, codebook):
    batch, dim, frames = x.shape
    ncodes = codebook.shape[0]
    nf = frames // _FB

    hi32 = lax.bitcast_convert_type(
        lax.bitcast_convert_type(codebook, jnp.uint32) & jnp.uint32(0xFFFF0000),
        jnp.float32)
    cb_hi = hi32.astype(jnp.bfloat16)
    cb_lo = ((codebook - hi32) * 512.0).astype(jnp.bfloat16)
    cb_cat = jnp.concatenate([cb_hi, cb_lo], axis=1)

    q, codes3, lossp = pl.pallas_call(
        _vq_body,
        grid=(batch, nf),
        in_specs=[
            pl.BlockSpec((1, dim, _FB), lambda b, f: (b, 0, f)),
            pl.BlockSpec((ncodes, dim), lambda b, f: (0, 0)),
            pl.BlockSpec((ncodes, 2 * dim), lambda b, f: (0, 0)),
        ],
        out_specs=[
            pl.BlockSpec((1, dim, _FB), lambda b, f: (b, 0, f)),
            pl.BlockSpec((1, 1, _FB), lambda b, f: (b, 0, f)),
            pl.BlockSpec((1, 1, 8, 128), lambda b, f: (b, f, 0, 0)),
        ],
        out_shape=[
            jax.ShapeDtypeStruct((batch, dim, frames), jnp.float32),
            jax.ShapeDtypeStruct((batch, 1, frames), jnp.int32),
            jax.ShapeDtypeStruct((batch, nf, 8, 128), jnp.float32),
        ],
        compiler_params=pltpu.CompilerParams(
            dimension_semantics=("parallel", "parallel"),
        ),
    )(x, codebook, cb_cat)

    codes = codes3.reshape(batch, frames)
    vq_loss = 1.25 * jnp.sum(lossp[:, :, 0, 0]) / (batch * dim * frames)
    return (q, codes, vq_loss)
